# Initial kernel scaffold; baseline (speedup 1.0000x reference)
import jax
import jax.numpy as jnp
from jax.experimental import pallas as pl


def kernel(y):
    raise NotImplementedError("write your pallas kernel here")


# speedup vs baseline: 23.0826x; 23.0826x over previous
import jax
import jax.numpy as jnp
from jax.experimental import pallas as pl

_DIM = 128
_LEVELS = 16
_CLIP = 3.0 / (_DIM ** 0.5)
_STEP = (2.0 * _CLIP) / (_LEVELS - 1)
_INV_STEP = 1.0 / _STEP


def _quant_body(y_ref, o_ref):
    y = y_ref[...]
    yc = jnp.clip(y, -_CLIP, _CLIP)
    t = (yc + _CLIP) * _INV_STEP
    k = jnp.floor(t + 0.5)
    o_ref[...] = k * _STEP - _CLIP


def kernel(y):
    n, d = y.shape
    blk = 8192
    grid = (n // blk,)
    return pl.pallas_call(
        _quant_body,
        out_shape=jax.ShapeDtypeStruct((n, d), y.dtype),
        grid=grid,
        in_specs=[pl.BlockSpec((blk, d), lambda i: (i, 0))],
        out_specs=pl.BlockSpec((blk, d), lambda i: (i, 0)),
    )(y)


# speedup vs baseline: 23.7027x; 1.0269x over previous
import jax
import jax.numpy as jnp
from jax.experimental import pallas as pl

_DIM = 128
_LEVELS = 16
_CLIP = 3.0 / (_DIM ** 0.5)
_STEP = (2.0 * _CLIP) / (_LEVELS - 1)
_INV_STEP = 1.0 / _STEP


def _quant_body(y_ref, o_ref):
    y = y_ref[...]
    yc = jnp.clip(y, -_CLIP, _CLIP)
    t = (yc + _CLIP) * _INV_STEP
    k = jnp.floor(t + 0.5)
    o_ref[...] = k * _STEP - _CLIP


def kernel(y):
    n, d = y.shape
    blk = 16384
    grid = (n // blk,)
    return pl.pallas_call(
        _quant_body,
        out_shape=jax.ShapeDtypeStruct((n, d), y.dtype),
        grid=grid,
        in_specs=[pl.BlockSpec((blk, d), lambda i: (i, 0))],
        out_specs=pl.BlockSpec((blk, d), lambda i: (i, 0)),
    )(y)
